# baseline (speedup 1.0000x reference)
import functools

import jax
import jax.numpy as jnp
from jax import lax
from jax.experimental import pallas as pl
from jax.experimental.pallas import tpu as pltpu
from jax.experimental.pallas import tpu_sc as plsc

NC = 2
NS = 16
NW = NC * NS

CHUNK = 8
NBUF = 4
LOOKAHEAD = 2


def _make_gather(vocab, dim, n):
    assert n % NW == 0
    b_per_w = n // NW
    assert b_per_w % CHUNK == 0
    n_chunks = b_per_w // CHUNK

    mesh = plsc.VectorSubcoreMesh(core_axis_name="c", subcore_axis_name="s")

    @functools.partial(
        pl.kernel,
        out_type=jax.ShapeDtypeStruct((n, dim), jnp.float32),
        mesh=mesh,
        scratch_types=[
            pltpu.VMEM((b_per_w,), jnp.int32),
            [pltpu.VMEM((CHUNK, dim), jnp.float32) for _ in range(NBUF)],
            [pltpu.SemaphoreType.DMA for _ in range(NBUF)],
            [pltpu.SemaphoreType.DMA for _ in range(NBUF)],
        ],
    )
    def gather(table_hbm, idx_hbm, out_hbm, idx_v, bufs, gsems, ssems):
        wid = lax.axis_index("s") * NC + lax.axis_index("c")
        base = wid * b_per_w
        pltpu.sync_copy(idx_hbm.at[pl.ds(base, b_per_w)], idx_v)

        def issue_gather(chunk, i):
            pltpu.async_copy(
                table_hbm.at[idx_v.at[pl.ds(chunk * CHUNK, CHUNK)]],
                bufs[i],
                gsems[i],
            )

        for c in range(LOOKAHEAD):
            issue_gather(c, c % NBUF)

        def body(c, _):
            nxt = c + LOOKAHEAD
            for i in range(NBUF):
                @pl.when((lax.rem(nxt, NBUF) == i) & (nxt < n_chunks))
                def _():
                    @pl.when(nxt >= NBUF)
                    def _():
                        pltpu.make_async_copy(
                            bufs[i], out_hbm.at[pl.ds(base, CHUNK)], ssems[i]
                        ).wait()
                    issue_gather(nxt, i)

            for i in range(NBUF):
                @pl.when(lax.rem(c, NBUF) == i)
                def _():
                    pltpu.make_async_copy(
                        table_hbm.at[pl.ds(0, CHUNK)], bufs[i], gsems[i]
                    ).wait()
                    pltpu.async_copy(
                        bufs[i], out_hbm.at[pl.ds(base + c * CHUNK, CHUNK)], ssems[i]
                    )

            return 0

        lax.fori_loop(0, n_chunks, body, 0)

        for i in range(min(NBUF, n_chunks)):
            pltpu.make_async_copy(
                bufs[i], out_hbm.at[pl.ds(base, CHUNK)], ssems[i]
            ).wait()

    return gather


def kernel(input_ids, weight):
    b, s = input_ids.shape
    vocab, dim = weight.shape
    idx = input_ids.reshape(-1).astype(jnp.int32)
    out = _make_gather(vocab, dim, idx.shape[0])(weight, idx)
    return out.reshape(b, s, dim)
